# baseline (speedup 1.0000x reference)
import functools

import jax
import jax.numpy as jnp
from jax import lax
from jax.experimental import pallas as pl
from jax.experimental.pallas import tpu as pltpu
from jax.experimental.pallas import tpu_sc as plsc

B = 4
S = 2048
D = 8192
ROWS = B * S
NC = 2
NS = 16
NW = NC * NS
R_PER_W = ROWS // NW
CH = 2
NCHUNK = R_PER_W // CH
NBUF = 6
DIST = 4
NROUND = (NCHUNK - 2) // NBUF
NTAIL = NCHUNK - NROUND * NBUF

_mesh = plsc.VectorSubcoreMesh(core_axis_name="c", subcore_axis_name="s")


@functools.partial(
    pl.kernel,
    mesh=_mesh,
    out_type=jax.ShapeDtypeStruct((ROWS, D), jnp.float32),
    scratch_types=[
        pltpu.VMEM((NCHUNK, CH), jnp.int32),
        pltpu.VMEM((NBUF, CH, D), jnp.float32),
        pltpu.SemaphoreType.DMA,
        pltpu.SemaphoreType.DMA,
        pltpu.SemaphoreType.DMA,
        pltpu.SemaphoreType.DMA,
        pltpu.SemaphoreType.DMA,
        pltpu.SemaphoreType.DMA,
        pltpu.SemaphoreType.DMA,
        pltpu.SemaphoreType.DMA,
        pltpu.SemaphoreType.DMA,
        pltpu.SemaphoreType.DMA,
        pltpu.SemaphoreType.DMA,
        pltpu.SemaphoreType.DMA,
    ],
)
def _gather_kernel(idx_hbm, table_hbm, out_hbm, idx_v, bufs, *sems):
    semg = sems[:NBUF]
    semo = sems[NBUF:]
    wid = lax.axis_index("s") * NC + lax.axis_index("c")
    base = wid * R_PER_W
    pltpu.sync_copy(idx_hbm.at[wid], idx_v)

    for b in range(DIST):
        pltpu.async_copy(table_hbm.at[idx_v.at[b]], bufs.at[b], semg[b])

    def body(i, carry):
        j0 = i * NBUF
        for b in range(NBUF):
            j = j0 + b
            bp = (b + DIST) % NBUF
            pltpu.make_async_copy(
                table_hbm.at[idx_v.at[j]], bufs.at[b], semg[b]).wait()
            pltpu.async_copy(
                bufs.at[b], out_hbm.at[pl.ds(base + j * CH, CH)], semo[b])
            @pl.when(j >= 2)
            def _():
                @pl.when(j + DIST < NCHUNK)
                def _():
                    pltpu.make_async_copy(
                        bufs.at[bp], out_hbm.at[pl.ds(base, CH)],
                        semo[bp]).wait()
            @pl.when(j + DIST < NCHUNK)
            def _():
                pltpu.async_copy(
                    table_hbm.at[idx_v.at[j + DIST]], bufs.at[bp], semg[bp])
        return carry

    lax.fori_loop(0, NROUND, body, 0)
    for t in range(NTAIL):
        j = NROUND * NBUF + t
        b = j % NBUF
        pltpu.make_async_copy(
            table_hbm.at[idx_v.at[j]], bufs.at[b], semg[b]).wait()
        pltpu.async_copy(
            bufs.at[b], out_hbm.at[pl.ds(base + j * CH, CH)], semo[b])
    for b in range(NBUF):
        pltpu.make_async_copy(
            bufs.at[b], out_hbm.at[pl.ds(base, CH)], semo[b]).wait()


def kernel(x, table):
    idx = x.reshape(NW, NCHUNK, CH).astype(jnp.int32)
    out = _gather_kernel(idx, table)
    return out.reshape(B, S, D)


# speedup vs baseline: 1.0022x; 1.0022x over previous
import functools

import jax
import jax.numpy as jnp
from jax import lax
from jax.experimental import pallas as pl
from jax.experimental.pallas import tpu as pltpu
from jax.experimental.pallas import tpu_sc as plsc

B = 4
S = 2048
D = 8192
ROWS = B * S
NC = 2
NS = 16
NW = NC * NS
R_PER_W = ROWS // NW
CH = 4
NCHUNK = R_PER_W // CH
NBUF = 3
DIST = 2
NROUND = (NCHUNK - 1) // NBUF
NTAIL = NCHUNK - NROUND * NBUF

_mesh = plsc.VectorSubcoreMesh(core_axis_name="c", subcore_axis_name="s")


@functools.partial(
    pl.kernel,
    mesh=_mesh,
    out_type=jax.ShapeDtypeStruct((ROWS, D), jnp.float32),
    scratch_types=[
        pltpu.VMEM((NCHUNK, CH), jnp.int32),
        pltpu.VMEM((NBUF, CH, D), jnp.float32),
        pltpu.SemaphoreType.DMA,
        pltpu.SemaphoreType.DMA,
        pltpu.SemaphoreType.DMA,
        pltpu.SemaphoreType.DMA,
        pltpu.SemaphoreType.DMA,
        pltpu.SemaphoreType.DMA,
    ],
)
def _gather_kernel(idx_hbm, table_hbm, out_hbm, idx_v, bufs, *sems):
    semg = sems[:NBUF]
    semo = sems[NBUF:]
    wid = lax.axis_index("s") * NC + lax.axis_index("c")
    base = wid * R_PER_W
    pltpu.sync_copy(idx_hbm.at[wid], idx_v)

    for b in range(DIST):
        pltpu.async_copy(table_hbm.at[idx_v.at[b]], bufs.at[b], semg[b])

    def body(i, carry):
        j0 = i * NBUF
        for b in range(NBUF):
            j = j0 + b
            bp = (b + DIST) % NBUF
            pltpu.make_async_copy(
                table_hbm.at[idx_v.at[j]], bufs.at[b], semg[b]).wait()
            pltpu.async_copy(
                bufs.at[b], out_hbm.at[pl.ds(base + j * CH, CH)], semo[b])
            @pl.when(j >= 1)
            def _():
                @pl.when(j + DIST < NCHUNK)
                def _():
                    pltpu.make_async_copy(
                        bufs.at[bp], out_hbm.at[pl.ds(base, CH)],
                        semo[bp]).wait()
            @pl.when(j + DIST < NCHUNK)
            def _():
                pltpu.async_copy(
                    table_hbm.at[idx_v.at[j + DIST]], bufs.at[bp], semg[bp])
        return carry

    lax.fori_loop(0, NROUND, body, 0)
    for t in range(NTAIL):
        j = NROUND * NBUF + t
        b = j % NBUF
        pltpu.make_async_copy(
            table_hbm.at[idx_v.at[j]], bufs.at[b], semg[b]).wait()
        pltpu.async_copy(
            bufs.at[b], out_hbm.at[pl.ds(base + j * CH, CH)], semo[b])
    for b in range(NBUF):
        pltpu.make_async_copy(
            bufs.at[b], out_hbm.at[pl.ds(base, CH)], semo[b]).wait()


def kernel(x, table):
    idx = x.reshape(NW, NCHUNK, CH).astype(jnp.int32)
    out = _gather_kernel(idx, table)
    return out.reshape(B, S, D)


# speedup vs baseline: 1.0101x; 1.0079x over previous
import functools

import jax
import jax.numpy as jnp
from jax import lax
from jax.experimental import pallas as pl
from jax.experimental.pallas import tpu as pltpu
from jax.experimental.pallas import tpu_sc as plsc

B = 4
S = 2048
D = 8192
ROWS = B * S
NC = 2
NS = 16
NW = NC * NS
R_PER_W = ROWS // NW
CH = 4
NCHUNK = R_PER_W // CH
NBUF = 3
NROUND = (NCHUNK - 1) // NBUF
NTAIL = NCHUNK - NROUND * NBUF

_mesh = plsc.VectorSubcoreMesh(core_axis_name="c", subcore_axis_name="s")


@functools.partial(
    pl.kernel,
    mesh=_mesh,
    out_type=jax.ShapeDtypeStruct((ROWS, D), jnp.float32),
    scratch_types=[
        pltpu.VMEM((NCHUNK, CH), jnp.int32),
        pltpu.VMEM((NBUF, CH, D), jnp.float32),
        pltpu.SemaphoreType.DMA,
        pltpu.SemaphoreType.DMA,
        pltpu.SemaphoreType.DMA,
        pltpu.SemaphoreType.DMA,
        pltpu.SemaphoreType.DMA,
        pltpu.SemaphoreType.DMA,
    ],
)
def _gather_kernel(idx_hbm, table_hbm, out_hbm, idx_v, bufs,
                   sg0, sg1, sg2, so0, so1, so2):
    semg = (sg0, sg1, sg2)
    semo = (so0, so1, so2)
    wid = lax.axis_index("s") * NC + lax.axis_index("c")
    base = wid * R_PER_W
    pltpu.sync_copy(idx_hbm.at[wid], idx_v)

    for b in range(NBUF):
        pltpu.async_copy(table_hbm.at[idx_v.at[b]], bufs.at[b], semg[b])

    def body(i, carry):
        j0 = i * NBUF
        for b in range(NBUF):
            j = j0 + b
            pltpu.make_async_copy(
                table_hbm.at[idx_v.at[j]], bufs.at[b], semg[b]).wait()
            pltpu.async_copy(
                bufs.at[b], out_hbm.at[pl.ds(base + j * CH, CH)], semo[b])
            pltpu.make_async_copy(
                bufs.at[b], out_hbm.at[pl.ds(base, CH)], semo[b]).wait()
            @pl.when(j + NBUF < NCHUNK)
            def _():
                pltpu.async_copy(
                    table_hbm.at[idx_v.at[j + NBUF]], bufs.at[b], semg[b])
        return carry

    lax.fori_loop(0, NROUND, body, 0)
    for t in range(NTAIL):
        j = NROUND * NBUF + t
        pltpu.make_async_copy(
            table_hbm.at[idx_v.at[j]], bufs.at[t], semg[t]).wait()
        pltpu.async_copy(
            bufs.at[t], out_hbm.at[pl.ds(base + j * CH, CH)], semo[t])
    for t in range(NTAIL):
        pltpu.make_async_copy(
            bufs.at[t], out_hbm.at[pl.ds(base, CH)], semo[t]).wait()


def kernel(x, table):
    idx = x.reshape(NW, NCHUNK, CH).astype(jnp.int32)
    out = _gather_kernel(idx, table)
    return out.reshape(B, S, D)
